# Initial kernel scaffold; baseline (speedup 1.0000x reference)
import jax
import jax.numpy as jnp
from jax.experimental import pallas as pl


def kernel(chn_llr, edge_row, edge_col, W1, b1, W2, b2):
    raise NotImplementedError("write your pallas kernel here")


# speedup vs baseline: 2.8808x; 2.8808x over previous
import math

import jax
import jax.numpy as jnp
from jax import lax
from jax.experimental import pallas as pl
from jax.experimental.pallas import tpu as pltpu
from jax.experimental.pallas import tpu_sc as plsc

N = 8192
M = 4096
E = 32768
B = 64
T = 10
LLR_CLIP = 15.0
_LO = -math.log(math.tanh(LLR_CLIP / 2))

NC = 2
NS = 16
BH = B // 2

_f32 = jnp.float32


def _sc_mesh():
    return plsc.VectorSubcoreMesh(
        core_axis_name="c", subcore_axis_name="s", num_cores=NC, num_subcores=NS
    )


_COL_CH = 128
_COL_NCH = 2048 // _COL_CH


def _sccol_body(c2v3, idx2d, chn3, colg3, out3, idx_s, val, acc,
                lsem, gsem):
    s = lax.axis_index("s")
    c = lax.axis_index("c")

    pltpu.sync_copy(chn3.at[c, pl.ds(s * 512, 512)], acc.at[pl.ds(s * 512, 512)])
    pltpu.sync_copy(idx2d.at[pl.ds(s * _COL_NCH, _COL_NCH)], idx_s)
    plsc.subcore_barrier()

    e_base = s * 2048

    pltpu.async_copy(c2v3.at[c, pl.ds(e_base, _COL_CH)], val.at[0], lsem.at[0])

    def scat_j(j, carry):
        b = j % 2

        @pl.when(j < _COL_NCH - 1)
        def _():
            pltpu.async_copy(
                c2v3.at[c, pl.ds(e_base + (j + 1) * _COL_CH, _COL_CH)],
                val.at[1 - b], lsem.at[1 - b])

        pltpu.make_async_copy(
            c2v3.at[c, pl.ds(e_base, _COL_CH)], val.at[b], lsem.at[b]).wait()
        pltpu.sync_copy(val.at[b], acc.at[idx_s.at[j]], add=True)
        return carry

    lax.fori_loop(0, _COL_NCH, scat_j, 0)
    plsc.subcore_barrier()

    pltpu.async_copy(acc.at[idx_s.at[0]], val.at[0], gsem.at[0])

    def gath_j(j, carry):
        b = j % 2

        @pl.when(j < _COL_NCH - 1)
        def _():
            pltpu.async_copy(acc.at[idx_s.at[j + 1]], val.at[1 - b],
                             gsem.at[1 - b])

        pltpu.make_async_copy(acc.at[idx_s.at[0]], val.at[b], gsem.at[b]).wait()
        pltpu.sync_copy(val.at[b],
                        colg3.at[c, pl.ds(e_base + j * _COL_CH, _COL_CH)])
        return carry

    lax.fori_loop(0, _COL_NCH, gath_j, 0)

    pltpu.sync_copy(acc.at[pl.ds(s * 512, 512)], out3.at[c, pl.ds(s * 512, 512)])


def _sccol(c2v3, ecol2d, chn3):
    fn = pl.kernel(
        _sccol_body,
        out_type=(
            jax.ShapeDtypeStruct((NC, E, BH), _f32),
            jax.ShapeDtypeStruct((NC, N, BH), _f32),
        ),
        mesh=_sc_mesh(),
        scratch_types=[
            pltpu.VMEM((_COL_NCH, _COL_CH), jnp.int32),
            pltpu.VMEM((2, _COL_CH, BH), _f32),
            pltpu.VMEM_SHARED((N, BH), _f32),
            pltpu.SemaphoreType.DMA((2,)),
            pltpu.SemaphoreType.DMA((2,)),
        ],
    )
    return fn(c2v3, ecol2d, chn3)


_ROW_CH = 128
_ROW_NCH = 2048 // _ROW_CH


def _sch_body(sl3, idx2d, zeros_pa, pag3,
              idx_s, val, gbuf, acc, lsem, gsem):
    s = lax.axis_index("s")
    c = lax.axis_index("c")

    pltpu.sync_copy(zeros_pa.at[pl.ds(s * 256, 256)], acc.at[pl.ds(s * 256, 256)])
    pltpu.sync_copy(idx2d.at[pl.ds(s * _ROW_NCH, _ROW_NCH)], idx_s)
    plsc.subcore_barrier()

    e_base = s * 2048

    pltpu.async_copy(sl3.at[c, pl.ds(e_base, _ROW_CH)], val.at[0], lsem.at[0])

    def scat_j(j, carry):
        b = j % 2

        @pl.when(j < _ROW_NCH - 1)
        def _():
            pltpu.async_copy(
                sl3.at[c, pl.ds(e_base + (j + 1) * _ROW_CH, _ROW_CH)],
                val.at[1 - b], lsem.at[1 - b])

        pltpu.make_async_copy(
            sl3.at[c, pl.ds(e_base, _ROW_CH)], val.at[b], lsem.at[b]).wait()
        pltpu.sync_copy(val.at[b], acc.at[idx_s.at[j]], add=True)
        return carry

    lax.fori_loop(0, _ROW_NCH, scat_j, 0)
    plsc.subcore_barrier()

    pltpu.async_copy(acc.at[idx_s.at[0]], gbuf.at[0], gsem.at[0])

    def gath_j(j, carry):
        b = j % 2

        @pl.when(j < _ROW_NCH - 1)
        def _():
            pltpu.async_copy(acc.at[idx_s.at[j + 1]], gbuf.at[1 - b],
                             gsem.at[1 - b])

        pltpu.make_async_copy(acc.at[idx_s.at[0]], gbuf.at[b], gsem.at[b]).wait()
        pltpu.sync_copy(gbuf.at[b],
                        pag3.at[c, pl.ds(e_base + j * _ROW_CH, _ROW_CH)])
        return carry

    lax.fori_loop(0, _ROW_NCH, gath_j, 0)


def _sch(sl3, erow2d, zeros_pa):
    fn = pl.kernel(
        _sch_body,
        out_type=jax.ShapeDtypeStruct((NC, E, B), _f32),
        mesh=_sc_mesh(),
        scratch_types=[
            pltpu.VMEM((_ROW_NCH, _ROW_CH), jnp.int32),
            pltpu.VMEM((2, _ROW_CH, B), _f32),
            pltpu.VMEM((2, _ROW_CH, B), _f32),
            pltpu.VMEM_SHARED((M, B), _f32),
            pltpu.SemaphoreType.DMA((2,)),
            pltpu.SemaphoreType.DMA((2,)),
        ],
    )
    return fn(sl3, erow2d, zeros_pa)


_EBLK = 4096


def _tcv_body(g_ref, v2c_ref, c2v_ref, colg_ref, v2c_o, sl_o):
    g = g_ref[...]
    v_new = colg_ref[...] - c2v_ref[...]
    v2c = (1.0 - g) * v2c_ref[...] + g * v_new
    v2c_o[...] = v2c
    lam = jnp.clip(v2c, -LLR_CLIP, LLR_CLIP)
    sign = (lam < 0).astype(_f32)
    abs_lam = jnp.clip(jnp.abs(lam), _LO, LLR_CLIP)
    lth = jnp.log(jnp.tanh(abs_lam * 0.5))
    sl_o[...] = jnp.concatenate([sign, lth], axis=-1)


def _tcv(g2, v2c3, c2v3, colg3):
    blk = lambda: pl.BlockSpec((1, _EBLK, BH), lambda h, i: (h, i, 0))
    blk64 = pl.BlockSpec((1, _EBLK, B), lambda h, i: (h, i, 0))
    outs = pl.pallas_call(
        _tcv_body,
        grid=(NC, E // _EBLK),
        in_specs=[pl.BlockSpec((1, 1, BH), lambda h, i: (h, 0, 0)),
                  blk(), blk(), blk()],
        out_specs=[blk(), blk64],
        out_shape=[jax.ShapeDtypeStruct((NC, E, BH), _f32),
                   jax.ShapeDtypeStruct((NC, E, B), _f32)],
    )(g2, v2c3, c2v3, colg3)
    return outs


def _tcb_body(g_ref, pag_ref, sl_ref, c2v_ref, c2v_o):
    g = g_ref[...]
    pag = pag_ref[...]
    sl = sl_ref[...]
    parity = pag[:, :, :BH] - sl[:, :, :BH]
    sgn = 1.0 - 2.0 * jnp.mod(parity, 2.0)
    amp = pag[:, :, BH:] - sl[:, :, BH:]
    x = jnp.exp(amp) * (1.0 - 1e-6)
    c_new = sgn * jnp.log((1.0 + x) / (1.0 - x))
    c2v_o[...] = (1.0 - g) * c2v_ref[...] + g * c_new


def _tcb(g2, pag3, sl3, c2v3):
    blk = lambda: pl.BlockSpec((1, _EBLK, BH), lambda h, i: (h, i, 0))
    blk64 = lambda: pl.BlockSpec((1, _EBLK, B), lambda h, i: (h, i, 0))
    out = pl.pallas_call(
        _tcb_body,
        grid=(NC, E // _EBLK),
        in_specs=[pl.BlockSpec((1, 1, BH), lambda h, i: (h, 0, 0)),
                  blk64(), blk64(), blk()],
        out_specs=blk(),
        out_shape=jax.ShapeDtypeStruct((NC, E, BH), _f32),
    )(g2, pag3, sl3, c2v3)
    return out


def _adapter(chn_llr, W1, b1, W2, b2):
    bm, bk = 512, 512
    nk = N // bk

    def mm1_body(w_ref, x_ref, bias_ref, o_ref, acc_ref):
        k = pl.program_id(1)

        @pl.when(k == 0)
        def _():
            acc_ref[...] = jnp.zeros_like(acc_ref)

        acc_ref[...] += jnp.dot(w_ref[...], x_ref[...],
                                preferred_element_type=_f32)

        @pl.when(k == nk - 1)
        def _():
            o_ref[...] = jnp.maximum(acc_ref[...] + bias_ref[...], 0.0)

    h = pl.pallas_call(
        mm1_body,
        grid=(N // bm, nk),
        in_specs=[
            pl.BlockSpec((bm, bk), lambda i, k: (i, k)),
            pl.BlockSpec((bk, B), lambda i, k: (k, 0)),
            pl.BlockSpec((bm, 1), lambda i, k: (i, 0)),
        ],
        out_specs=pl.BlockSpec((bm, B), lambda i, k: (i, 0)),
        out_shape=jax.ShapeDtypeStruct((N, B), _f32),
        scratch_shapes=[pltpu.VMEM((bm, B), _f32)],
    )(W1, chn_llr, b1.reshape(N, 1))

    W2p = jnp.zeros((8, N), _f32).at[:3].set(W2)
    b2p = jnp.zeros((8, 1), _f32).at[:3, 0].set(b2)

    def mm2_body(w_ref, h_ref, bias_ref, o_ref):
        o_ref[...] = jax.nn.sigmoid(
            jnp.dot(w_ref[...], h_ref[...], preferred_element_type=_f32)
            + bias_ref[...]
        )

    ada = pl.pallas_call(
        mm2_body,
        out_shape=jax.ShapeDtypeStruct((8, B), _f32),
    )(W2p, h, b2p)
    return ada[0:1, :]


def kernel(chn_llr, edge_row, edge_col, W1, b1, W2, b2):
    ecol2d = edge_col.astype(jnp.int32).reshape(E // _COL_CH, _COL_CH)
    erow2d = edge_row.astype(jnp.int32).reshape(E // _ROW_CH, _ROW_CH)

    gamma = _adapter(chn_llr, W1, b1, W2, b2)
    g2 = jnp.stack([gamma[0, :BH], gamma[0, BH:]]).reshape(NC, 1, BH)

    chn3 = jnp.stack([chn_llr[:, :BH], chn_llr[:, BH:]])
    zeros_pa = jnp.zeros((M, B), _f32)
    c2v3 = jnp.zeros((NC, E, BH), _f32)
    v2c3 = jnp.zeros((NC, E, BH), _f32)

    colg3, _ = _sccol(c2v3, ecol2d, chn3)

    outs = []
    for _t in range(T):
        v2c3, sl3 = _tcv(g2, v2c3, c2v3, colg3)
        pag3 = _sch(sl3, erow2d, zeros_pa)
        c2v3 = _tcb(g2, pag3, sl3, c2v3)
        colg3, out3 = _sccol(c2v3, ecol2d, chn3)
        outs.append(out3)
    out = jnp.stack(outs)
    return jnp.concatenate([out[:, 0], out[:, 1]], axis=-1)


# speedup vs baseline: 4.9529x; 1.7193x over previous
import math

import jax
import jax.numpy as jnp
from jax import lax
from jax.experimental import pallas as pl
from jax.experimental.pallas import tpu as pltpu
from jax.experimental.pallas import tpu_sc as plsc

N = 8192
M = 4096
E = 32768
B = 64
T = 10
LLR_CLIP = 15.0
_LO = -math.log(math.tanh(LLR_CLIP / 2))

NC = 2
NS = 16
BH = B // 2

_f32 = jnp.float32


def _sc_mesh():
    return plsc.VectorSubcoreMesh(
        core_axis_name="c", subcore_axis_name="s", num_cores=NC, num_subcores=NS
    )


_SC_PARAMS = dict(
    compiler_params=pltpu.CompilerParams(use_tc_tiling_on_sc=False),
)

_NCH = 16


def _sccol_body(c2v3, idx2d, chn3, colg3, out3, idx_s, val, gbuf, acc,
                lsem, gsem):
    s = lax.axis_index("s")
    c = lax.axis_index("c")

    pltpu.sync_copy(chn3.at[c, pl.ds(s * 512, 512)], acc.at[pl.ds(s * 512, 512)])
    pltpu.sync_copy(idx2d.at[pl.ds(s * _NCH, _NCH)], idx_s)
    plsc.subcore_barrier()

    e_base = s * 2048

    pltpu.async_copy(c2v3.at[c, pl.ds(e_base, 128)], val.at[0], lsem.at[0])

    def scat_j(j, carry):
        b = j % 2

        @pl.when(j < _NCH - 1)
        def _():
            pltpu.async_copy(
                c2v3.at[c, pl.ds(e_base + (j + 1) * 128, 128)],
                val.at[1 - b], lsem.at[1 - b])

        pltpu.make_async_copy(
            c2v3.at[c, pl.ds(e_base, 128)], val.at[b], lsem.at[b]).wait()
        pltpu.sync_copy(val.at[b], acc.at[idx_s.at[j]], add=True)
        return carry

    lax.fori_loop(0, _NCH, scat_j, 0)
    plsc.subcore_barrier()

    pltpu.async_copy(acc.at[idx_s.at[0]], gbuf.at[0], gsem.at[0])

    def gath_j(j, carry):
        b = j % 2

        @pl.when(j < _NCH - 1)
        def _():
            pltpu.async_copy(acc.at[idx_s.at[j + 1]], gbuf.at[1 - b],
                             gsem.at[1 - b])

        pltpu.make_async_copy(acc.at[idx_s.at[0]], gbuf.at[b], gsem.at[b]).wait()
        pltpu.sync_copy(gbuf.at[b],
                        colg3.at[c, pl.ds(e_base + j * 128, 128)])
        return carry

    lax.fori_loop(0, _NCH, gath_j, 0)

    pltpu.sync_copy(acc.at[pl.ds(s * 512, 512)], out3.at[c, pl.ds(s * 512, 512)])


def _sccol(c2v_p, ecol2d, chn_p):
    fn = pl.kernel(
        _sccol_body,
        out_type=(
            jax.ShapeDtypeStruct((NC, E, BH), _f32),
            jax.ShapeDtypeStruct((NC, N, BH), _f32),
        ),
        mesh=_sc_mesh(),
        scratch_types=[
            pltpu.VMEM((_NCH, 128), jnp.int32),
            pltpu.VMEM((2, 128, BH), _f32),
            pltpu.VMEM((2, 128, BH), _f32),
            pltpu.VMEM_SHARED((N, BH), _f32),
            pltpu.SemaphoreType.DMA((2,)),
            pltpu.SemaphoreType.DMA((2,)),
        ],
        **_SC_PARAMS,
    )
    colg, out = fn(c2v_p.reshape(NC, E, BH), ecol2d, chn_p.reshape(NC, N, BH))
    return colg.reshape(NC, E // 4, 128), out.reshape(NC, N // 4, 128)


def _sch_body(sign3, lth3, idx2d, zeros_m, parg3, ampg3,
              idx_s, val_s, val_a, acc_p, acc_a, ls, la):
    s = lax.axis_index("s")
    c = lax.axis_index("c")

    pltpu.sync_copy(zeros_m.at[pl.ds(s * 256, 256)], acc_p.at[pl.ds(s * 256, 256)])
    pltpu.sync_copy(zeros_m.at[pl.ds(s * 256, 256)], acc_a.at[pl.ds(s * 256, 256)])
    pltpu.sync_copy(idx2d.at[pl.ds(s * _NCH, _NCH)], idx_s)
    plsc.subcore_barrier()

    e_base = s * 2048

    pltpu.async_copy(sign3.at[c, pl.ds(e_base, 128)], val_s.at[0], ls.at[0])
    pltpu.async_copy(lth3.at[c, pl.ds(e_base, 128)], val_a.at[0], la.at[0])

    def scat_j(j, carry):
        b = j % 2

        @pl.when(j < _NCH - 1)
        def _():
            pltpu.async_copy(sign3.at[c, pl.ds(e_base + (j + 1) * 128, 128)],
                             val_s.at[1 - b], ls.at[1 - b])
            pltpu.async_copy(lth3.at[c, pl.ds(e_base + (j + 1) * 128, 128)],
                             val_a.at[1 - b], la.at[1 - b])

        pltpu.make_async_copy(
            sign3.at[c, pl.ds(e_base, 128)], val_s.at[b], ls.at[b]).wait()
        pltpu.sync_copy(val_s.at[b], acc_p.at[idx_s.at[j]], add=True)
        pltpu.make_async_copy(
            lth3.at[c, pl.ds(e_base, 128)], val_a.at[b], la.at[b]).wait()
        pltpu.sync_copy(val_a.at[b], acc_a.at[idx_s.at[j]], add=True)
        return carry

    lax.fori_loop(0, _NCH, scat_j, 0)
    plsc.subcore_barrier()

    pltpu.async_copy(acc_p.at[idx_s.at[0]], val_s.at[0], ls.at[0])
    pltpu.async_copy(acc_a.at[idx_s.at[0]], val_a.at[0], la.at[0])

    def gath_j(j, carry):
        b = j % 2

        @pl.when(j < _NCH - 1)
        def _():
            pltpu.async_copy(acc_p.at[idx_s.at[j + 1]], val_s.at[1 - b],
                             ls.at[1 - b])
            pltpu.async_copy(acc_a.at[idx_s.at[j + 1]], val_a.at[1 - b],
                             la.at[1 - b])

        pltpu.make_async_copy(acc_p.at[idx_s.at[0]], val_s.at[b],
                              ls.at[b]).wait()
        pltpu.make_async_copy(acc_a.at[idx_s.at[0]], val_a.at[b],
                              la.at[b]).wait()
        pltpu.sync_copy(val_s.at[b], parg3.at[c, pl.ds(e_base + j * 128, 128)])
        pltpu.sync_copy(val_a.at[b], ampg3.at[c, pl.ds(e_base + j * 128, 128)])
        return carry

    lax.fori_loop(0, _NCH, gath_j, 0)


def _sch(sign_p, lth_p, erow2d, zeros_m):
    fn = pl.kernel(
        _sch_body,
        out_type=(
            jax.ShapeDtypeStruct((NC, E, BH), _f32),
            jax.ShapeDtypeStruct((NC, E, BH), _f32),
        ),
        mesh=_sc_mesh(),
        scratch_types=[
            pltpu.VMEM((_NCH, 128), jnp.int32),
            pltpu.VMEM((2, 128, BH), _f32),
            pltpu.VMEM((2, 128, BH), _f32),
            pltpu.VMEM_SHARED((M, BH), _f32),
            pltpu.VMEM_SHARED((M, BH), _f32),
            pltpu.SemaphoreType.DMA((2,)),
            pltpu.SemaphoreType.DMA((2,)),
        ],
        **_SC_PARAMS,
    )
    parg, ampg = fn(sign_p.reshape(NC, E, BH), lth_p.reshape(NC, E, BH),
                    erow2d, zeros_m)
    return parg.reshape(NC, E // 4, 128), ampg.reshape(NC, E // 4, 128)


_EROWS = E // 4
_EBLK = 2048


def _tcv_body(g_ref, v2c_ref, c2v_ref, colg_ref, v2c_o, sign_o, lth_o):
    g = g_ref[...]
    v_new = colg_ref[...] - c2v_ref[...]
    v2c = (1.0 - g) * v2c_ref[...] + g * v_new
    v2c_o[...] = v2c
    lam = jnp.clip(v2c, -LLR_CLIP, LLR_CLIP)
    sign_o[...] = (lam < 0).astype(_f32)
    abs_lam = jnp.clip(jnp.abs(lam), _LO, LLR_CLIP)
    lth_o[...] = jnp.log(jnp.tanh(abs_lam * 0.5))


def _tcv(g2, v2c_p, c2v_p, colg_p):
    blk = lambda: pl.BlockSpec((1, _EBLK, 128), lambda h, i: (h, i, 0))
    return pl.pallas_call(
        _tcv_body,
        grid=(NC, _EROWS // _EBLK),
        in_specs=[pl.BlockSpec((1, 1, 128), lambda h, i: (h, 0, 0)),
                  blk(), blk(), blk()],
        out_specs=[blk(), blk(), blk()],
        out_shape=[jax.ShapeDtypeStruct((NC, _EROWS, 128), _f32)] * 3,
    )(g2, v2c_p, c2v_p, colg_p)


def _tcb_body(g_ref, parg_ref, ampg_ref, sign_ref, lth_ref, c2v_ref, c2v_o):
    g = g_ref[...]
    parity = parg_ref[...] - sign_ref[...]
    sgn = 1.0 - 2.0 * jnp.mod(parity, 2.0)
    amp = ampg_ref[...] - lth_ref[...]
    x = jnp.exp(amp) * (1.0 - 1e-6)
    c_new = sgn * jnp.log((1.0 + x) / (1.0 - x))
    c2v_o[...] = (1.0 - g) * c2v_ref[...] + g * c_new


def _tcb(g2, parg_p, ampg_p, sign_p, lth_p, c2v_p):
    blk = lambda: pl.BlockSpec((1, _EBLK, 128), lambda h, i: (h, i, 0))
    return pl.pallas_call(
        _tcb_body,
        grid=(NC, _EROWS // _EBLK),
        in_specs=[pl.BlockSpec((1, 1, 128), lambda h, i: (h, 0, 0))] + [blk()] * 5,
        out_specs=blk(),
        out_shape=jax.ShapeDtypeStruct((NC, _EROWS, 128), _f32),
    )(g2, parg_p, ampg_p, sign_p, lth_p, c2v_p)


def _adapter(chn_llr, W1, b1, W2, b2):
    bm, bk = 512, 512
    nk = N // bk

    def mm1_body(w_ref, x_ref, bias_ref, o_ref, acc_ref):
        k = pl.program_id(1)

        @pl.when(k == 0)
        def _():
            acc_ref[...] = jnp.zeros_like(acc_ref)

        acc_ref[...] += jnp.dot(w_ref[...].astype(jnp.bfloat16),
                                x_ref[...].astype(jnp.bfloat16),
                                preferred_element_type=_f32)

        @pl.when(k == nk - 1)
        def _():
            o_ref[...] = jnp.maximum(acc_ref[...] + bias_ref[...], 0.0)

    h = pl.pallas_call(
        mm1_body,
        grid=(N // bm, nk),
        in_specs=[
            pl.BlockSpec((bm, bk), lambda i, k: (i, k)),
            pl.BlockSpec((bk, B), lambda i, k: (k, 0)),
            pl.BlockSpec((bm, 1), lambda i, k: (i, 0)),
        ],
        out_specs=pl.BlockSpec((bm, B), lambda i, k: (i, 0)),
        out_shape=jax.ShapeDtypeStruct((N, B), _f32),
        scratch_shapes=[pltpu.VMEM((bm, B), _f32)],
    )(W1, chn_llr, b1.reshape(N, 1))

    W2p = jnp.zeros((8, N), _f32).at[:3].set(W2)
    b2p = jnp.zeros((8, 1), _f32).at[:3, 0].set(b2)

    def mm2_body(w_ref, h_ref, bias_ref, o_ref):
        o_ref[...] = jax.nn.sigmoid(
            jnp.dot(w_ref[...], h_ref[...], preferred_element_type=_f32)
            + bias_ref[...]
        )

    ada = pl.pallas_call(
        mm2_body,
        out_shape=jax.ShapeDtypeStruct((8, B), _f32),
    )(W2p, h, b2p)
    return ada[0:1, :]


def kernel(chn_llr, edge_row, edge_col, W1, b1, W2, b2):
    ecol2d = edge_col.astype(jnp.int32).reshape(E // 128, 128)
    erow2d = edge_row.astype(jnp.int32).reshape(E // 128, 128)

    gamma = _adapter(chn_llr, W1, b1, W2, b2)
    g2 = jnp.stack([jnp.tile(gamma[0, :BH], 4),
                    jnp.tile(gamma[0, BH:], 4)]).reshape(NC, 1, 128)

    chn_p = jnp.stack([chn_llr[:, :BH].reshape(N // 4, 128),
                       chn_llr[:, BH:].reshape(N // 4, 128)])
    zeros_m = jnp.zeros((M, BH), _f32)
    c2v_p = jnp.zeros((NC, _EROWS, 128), _f32)
    v2c_p = jnp.zeros((NC, _EROWS, 128), _f32)

    colg_p, _ = _sccol(c2v_p, ecol2d, chn_p)

    outs = []
    for _t in range(T):
        v2c_p, sign_p, lth_p = _tcv(g2, v2c_p, c2v_p, colg_p)
        parg_p, ampg_p = _sch(sign_p, lth_p, erow2d, zeros_m)
        c2v_p = _tcb(g2, parg_p, ampg_p, sign_p, lth_p, c2v_p)
        colg_p, out_p = _sccol(c2v_p, ecol2d, chn_p)
        outs.append(out_p)
    out = jnp.stack(outs)
    out = out.reshape(T, NC, N, BH)
    return jnp.concatenate([out[:, 0], out[:, 1]], axis=-1)


# speedup vs baseline: 5.5658x; 1.1237x over previous
import math

import jax
import jax.numpy as jnp
from jax import lax
from jax.experimental import pallas as pl
from jax.experimental.pallas import tpu as pltpu
from jax.experimental.pallas import tpu_sc as plsc

N = 8192
M = 4096
E = 32768
B = 64
T = 10
LLR_CLIP = 15.0
_LO = -math.log(math.tanh(LLR_CLIP / 2))

NC = 2
NS = 16
BH = B // 2

_f32 = jnp.float32


def _sc_mesh():
    return plsc.VectorSubcoreMesh(
        core_axis_name="c", subcore_axis_name="s", num_cores=NC, num_subcores=NS
    )


_SC_PARAMS = dict(
    compiler_params=pltpu.CompilerParams(use_tc_tiling_on_sc=False),
)

_NCH = 16


def _sccol_body(c2v3, idx2d, chn3, colg3, out3, idx_s, val, gbuf, acc,
                lsem, gsem):
    s = lax.axis_index("s")
    c = lax.axis_index("c")

    pltpu.sync_copy(chn3.at[c, pl.ds(s * 512, 512)], acc.at[pl.ds(s * 512, 512)])
    pltpu.sync_copy(idx2d.at[pl.ds(s * _NCH, _NCH)], idx_s)
    plsc.subcore_barrier()

    e_base = s * 2048

    pltpu.async_copy(c2v3.at[c, pl.ds(e_base, 128)], val.at[0], lsem.at[0])

    def scat_j(j, carry):
        b = j % 2

        @pl.when(j < _NCH - 1)
        def _():
            pltpu.async_copy(
                c2v3.at[c, pl.ds(e_base + (j + 1) * 128, 128)],
                val.at[1 - b], lsem.at[1 - b])

        pltpu.make_async_copy(
            c2v3.at[c, pl.ds(e_base, 128)], val.at[b], lsem.at[b]).wait()
        pltpu.sync_copy(val.at[b], acc.at[idx_s.at[j]], add=True)
        return carry

    lax.fori_loop(0, _NCH, scat_j, 0)
    plsc.subcore_barrier()

    pltpu.async_copy(acc.at[idx_s.at[0]], gbuf.at[0], gsem.at[0])

    def gath_j(j, carry):
        b = j % 2

        @pl.when(j < _NCH - 1)
        def _():
            pltpu.async_copy(acc.at[idx_s.at[j + 1]], gbuf.at[1 - b],
                             gsem.at[1 - b])

        pltpu.make_async_copy(acc.at[idx_s.at[0]], gbuf.at[b], gsem.at[b]).wait()
        pltpu.sync_copy(gbuf.at[b],
                        colg3.at[c, pl.ds(e_base + j * 128, 128)])
        return carry

    lax.fori_loop(0, _NCH, gath_j, 0)

    pltpu.sync_copy(acc.at[pl.ds(s * 512, 512)], out3.at[c, pl.ds(s * 512, 512)])


def _sccol_first_body(idx2d, chn3, colg3, idx_s, gbuf, acc, gsem):
    s = lax.axis_index("s")
    c = lax.axis_index("c")

    pltpu.sync_copy(chn3.at[c, pl.ds(s * 512, 512)], acc.at[pl.ds(s * 512, 512)])
    pltpu.sync_copy(idx2d.at[pl.ds(s * _NCH, _NCH)], idx_s)
    plsc.subcore_barrier()

    e_base = s * 2048
    pltpu.async_copy(acc.at[idx_s.at[0]], gbuf.at[0], gsem.at[0])

    def gath_j(j, carry):
        b = j % 2

        @pl.when(j < _NCH - 1)
        def _():
            pltpu.async_copy(acc.at[idx_s.at[j + 1]], gbuf.at[1 - b],
                             gsem.at[1 - b])

        pltpu.make_async_copy(acc.at[idx_s.at[0]], gbuf.at[b], gsem.at[b]).wait()
        pltpu.sync_copy(gbuf.at[b],
                        colg3.at[c, pl.ds(e_base + j * 128, 128)])
        return carry

    lax.fori_loop(0, _NCH, gath_j, 0)


def _sccol_first(ecol2d, chn_p):
    fn = pl.kernel(
        _sccol_first_body,
        out_type=jax.ShapeDtypeStruct((NC, E, BH), _f32),
        mesh=_sc_mesh(),
        scratch_types=[
            pltpu.VMEM((_NCH, 128), jnp.int32),
            pltpu.VMEM((2, 128, BH), _f32),
            pltpu.VMEM_SHARED((N, BH), _f32),
            pltpu.SemaphoreType.DMA((2,)),
        ],
        **_SC_PARAMS,
    )
    colg = fn(ecol2d, chn_p.reshape(NC, N, BH))
    return colg.reshape(NC, E // 4, 128)


def _sccol_last_body(c2v3, idx2d, chn3, out3, idx_s, val, acc, lsem):
    s = lax.axis_index("s")
    c = lax.axis_index("c")

    pltpu.sync_copy(chn3.at[c, pl.ds(s * 512, 512)], acc.at[pl.ds(s * 512, 512)])
    pltpu.sync_copy(idx2d.at[pl.ds(s * _NCH, _NCH)], idx_s)
    plsc.subcore_barrier()

    e_base = s * 2048
    pltpu.async_copy(c2v3.at[c, pl.ds(e_base, 128)], val.at[0], lsem.at[0])

    def scat_j(j, carry):
        b = j % 2

        @pl.when(j < _NCH - 1)
        def _():
            pltpu.async_copy(
                c2v3.at[c, pl.ds(e_base + (j + 1) * 128, 128)],
                val.at[1 - b], lsem.at[1 - b])

        pltpu.make_async_copy(
            c2v3.at[c, pl.ds(e_base, 128)], val.at[b], lsem.at[b]).wait()
        pltpu.sync_copy(val.at[b], acc.at[idx_s.at[j]], add=True)
        return carry

    lax.fori_loop(0, _NCH, scat_j, 0)
    plsc.subcore_barrier()

    pltpu.sync_copy(acc.at[pl.ds(s * 512, 512)], out3.at[c, pl.ds(s * 512, 512)])


def _sccol_last(c2v_p, ecol2d, chn_p):
    fn = pl.kernel(
        _sccol_last_body,
        out_type=jax.ShapeDtypeStruct((NC, N, BH), _f32),
        mesh=_sc_mesh(),
        scratch_types=[
            pltpu.VMEM((_NCH, 128), jnp.int32),
            pltpu.VMEM((2, 128, BH), _f32),
            pltpu.VMEM_SHARED((N, BH), _f32),
            pltpu.SemaphoreType.DMA((2,)),
        ],
        **_SC_PARAMS,
    )
    out = fn(c2v_p.reshape(NC, E, BH), ecol2d, chn_p.reshape(NC, N, BH))
    return out.reshape(NC, N // 4, 128)


def _sccol(c2v_p, ecol2d, chn_p):
    fn = pl.kernel(
        _sccol_body,
        out_type=(
            jax.ShapeDtypeStruct((NC, E, BH), _f32),
            jax.ShapeDtypeStruct((NC, N, BH), _f32),
        ),
        mesh=_sc_mesh(),
        scratch_types=[
            pltpu.VMEM((_NCH, 128), jnp.int32),
            pltpu.VMEM((2, 128, BH), _f32),
            pltpu.VMEM((2, 128, BH), _f32),
            pltpu.VMEM_SHARED((N, BH), _f32),
            pltpu.SemaphoreType.DMA((2,)),
            pltpu.SemaphoreType.DMA((2,)),
        ],
        **_SC_PARAMS,
    )
    colg, out = fn(c2v_p.reshape(NC, E, BH), ecol2d, chn_p.reshape(NC, N, BH))
    return colg.reshape(NC, E // 4, 128), out.reshape(NC, N // 4, 128)


def _sch_body(sign3, lth3, idx2d, zeros_m, parg3, ampg3,
              idx_s, val_s, val_a, acc_p, acc_a, ls, la):
    s = lax.axis_index("s")
    c = lax.axis_index("c")

    pltpu.sync_copy(zeros_m.at[pl.ds(s * 256, 256)], acc_p.at[pl.ds(s * 256, 256)])
    pltpu.sync_copy(zeros_m.at[pl.ds(s * 256, 256)], acc_a.at[pl.ds(s * 256, 256)])
    pltpu.sync_copy(idx2d.at[pl.ds(s * _NCH, _NCH)], idx_s)
    plsc.subcore_barrier()

    e_base = s * 2048

    pltpu.async_copy(sign3.at[c, pl.ds(e_base, 128)], val_s.at[0], ls.at[0])
    pltpu.async_copy(lth3.at[c, pl.ds(e_base, 128)], val_a.at[0], la.at[0])

    def scat_j(j, carry):
        b = j % 2

        @pl.when(j < _NCH - 1)
        def _():
            pltpu.async_copy(sign3.at[c, pl.ds(e_base + (j + 1) * 128, 128)],
                             val_s.at[1 - b], ls.at[1 - b])
            pltpu.async_copy(lth3.at[c, pl.ds(e_base + (j + 1) * 128, 128)],
                             val_a.at[1 - b], la.at[1 - b])

        pltpu.make_async_copy(
            sign3.at[c, pl.ds(e_base, 128)], val_s.at[b], ls.at[b]).wait()
        pltpu.sync_copy(val_s.at[b], acc_p.at[idx_s.at[j]], add=True)
        pltpu.make_async_copy(
            lth3.at[c, pl.ds(e_base, 128)], val_a.at[b], la.at[b]).wait()
        pltpu.sync_copy(val_a.at[b], acc_a.at[idx_s.at[j]], add=True)
        return carry

    lax.fori_loop(0, _NCH, scat_j, 0)
    plsc.subcore_barrier()

    pltpu.async_copy(acc_p.at[idx_s.at[0]], val_s.at[0], ls.at[0])
    pltpu.async_copy(acc_a.at[idx_s.at[0]], val_a.at[0], la.at[0])

    def gath_j(j, carry):
        b = j % 2

        @pl.when(j < _NCH - 1)
        def _():
            pltpu.async_copy(acc_p.at[idx_s.at[j + 1]], val_s.at[1 - b],
                             ls.at[1 - b])
            pltpu.async_copy(acc_a.at[idx_s.at[j + 1]], val_a.at[1 - b],
                             la.at[1 - b])

        pltpu.make_async_copy(acc_p.at[idx_s.at[0]], val_s.at[b],
                              ls.at[b]).wait()
        pltpu.make_async_copy(acc_a.at[idx_s.at[0]], val_a.at[b],
                              la.at[b]).wait()
        pltpu.sync_copy(val_s.at[b], parg3.at[c, pl.ds(e_base + j * 128, 128)])
        pltpu.sync_copy(val_a.at[b], ampg3.at[c, pl.ds(e_base + j * 128, 128)])
        return carry

    lax.fori_loop(0, _NCH, gath_j, 0)


def _sch(sign_p, lth_p, erow2d, zeros_m):
    fn = pl.kernel(
        _sch_body,
        out_type=(
            jax.ShapeDtypeStruct((NC, E, BH), _f32),
            jax.ShapeDtypeStruct((NC, E, BH), _f32),
        ),
        mesh=_sc_mesh(),
        scratch_types=[
            pltpu.VMEM((_NCH, 128), jnp.int32),
            pltpu.VMEM((2, 128, BH), _f32),
            pltpu.VMEM((2, 128, BH), _f32),
            pltpu.VMEM_SHARED((M, BH), _f32),
            pltpu.VMEM_SHARED((M, BH), _f32),
            pltpu.SemaphoreType.DMA((2,)),
            pltpu.SemaphoreType.DMA((2,)),
        ],
        **_SC_PARAMS,
    )
    parg, ampg = fn(sign_p.reshape(NC, E, BH), lth_p.reshape(NC, E, BH),
                    erow2d, zeros_m)
    return parg.reshape(NC, E // 4, 128), ampg.reshape(NC, E // 4, 128)


_EROWS = E // 4
_EBLK = 2048


def _tcv_body(g_ref, v2c_ref, c2v_ref, colg_ref, v2c_o, sign_o, lth_o):
    g = g_ref[...]
    v_new = colg_ref[...] - c2v_ref[...]
    v2c = (1.0 - g) * v2c_ref[...] + g * v_new
    v2c_o[...] = v2c
    lam = jnp.clip(v2c, -LLR_CLIP, LLR_CLIP)
    sign_o[...] = (lam < 0).astype(_f32)
    abs_lam = jnp.clip(jnp.abs(lam), _LO, LLR_CLIP)
    lth_o[...] = jnp.log(jnp.tanh(abs_lam * 0.5))


def _tcv(g2, v2c_p, c2v_p, colg_p):
    blk = lambda: pl.BlockSpec((1, _EBLK, 128), lambda h, i: (h, i, 0))
    return pl.pallas_call(
        _tcv_body,
        grid=(NC, _EROWS // _EBLK),
        in_specs=[pl.BlockSpec((1, 1, 128), lambda h, i: (h, 0, 0)),
                  blk(), blk(), blk()],
        out_specs=[blk(), blk(), blk()],
        out_shape=[jax.ShapeDtypeStruct((NC, _EROWS, 128), _f32)] * 3,
    )(g2, v2c_p, c2v_p, colg_p)


def _tcb_body(g_ref, parg_ref, ampg_ref, sign_ref, lth_ref, c2v_ref, c2v_o):
    g = g_ref[...]
    parity = parg_ref[...] - sign_ref[...]
    sgn = 1.0 - 2.0 * jnp.mod(parity, 2.0)
    amp = ampg_ref[...] - lth_ref[...]
    x = jnp.exp(amp) * (1.0 - 1e-6)
    c_new = sgn * jnp.log((1.0 + x) / (1.0 - x))
    c2v_o[...] = (1.0 - g) * c2v_ref[...] + g * c_new


def _tcb(g2, parg_p, ampg_p, sign_p, lth_p, c2v_p):
    blk = lambda: pl.BlockSpec((1, _EBLK, 128), lambda h, i: (h, i, 0))
    return pl.pallas_call(
        _tcb_body,
        grid=(NC, _EROWS // _EBLK),
        in_specs=[pl.BlockSpec((1, 1, 128), lambda h, i: (h, 0, 0))] + [blk()] * 5,
        out_specs=blk(),
        out_shape=jax.ShapeDtypeStruct((NC, _EROWS, 128), _f32),
    )(g2, parg_p, ampg_p, sign_p, lth_p, c2v_p)


def _adapter(chn_llr, W1, b1, W2, b2):
    bm, bk = 1024, 2048
    nk = N // bk

    def mm1_body(w_ref, x_ref, bias_ref, o_ref, acc_ref):
        k = pl.program_id(1)

        @pl.when(k == 0)
        def _():
            acc_ref[...] = jnp.zeros_like(acc_ref)

        acc_ref[...] += jnp.dot(w_ref[...], x_ref[...],
                                preferred_element_type=_f32)

        @pl.when(k == nk - 1)
        def _():
            o_ref[...] = jnp.maximum(acc_ref[...] + bias_ref[...], 0.0)

    h = pl.pallas_call(
        mm1_body,
        grid=(N // bm, nk),
        in_specs=[
            pl.BlockSpec((bm, bk), lambda i, k: (i, k)),
            pl.BlockSpec((bk, B), lambda i, k: (k, 0)),
            pl.BlockSpec((bm, 1), lambda i, k: (i, 0)),
        ],
        out_specs=pl.BlockSpec((bm, B), lambda i, k: (i, 0)),
        out_shape=jax.ShapeDtypeStruct((N, B), _f32),
        scratch_shapes=[pltpu.VMEM((bm, B), _f32)],
    )(W1, chn_llr, b1.reshape(N, 1))

    W2p = jnp.zeros((8, N), _f32).at[:3].set(W2)
    b2p = jnp.zeros((8, 1), _f32).at[:3, 0].set(b2)

    def mm2_body(w_ref, h_ref, bias_ref, o_ref):
        o_ref[...] = jax.nn.sigmoid(
            jnp.dot(w_ref[...], h_ref[...], preferred_element_type=_f32)
            + bias_ref[...]
        )

    ada = pl.pallas_call(
        mm2_body,
        out_shape=jax.ShapeDtypeStruct((8, B), _f32),
    )(W2p, h, b2p)
    return ada[0:1, :]


def kernel(chn_llr, edge_row, edge_col, W1, b1, W2, b2):
    ecol2d = edge_col.astype(jnp.int32).reshape(E // 128, 128)
    erow2d = edge_row.astype(jnp.int32).reshape(E // 128, 128)

    gamma = _adapter(chn_llr, W1, b1, W2, b2)
    g2 = jnp.stack([jnp.tile(gamma[0, :BH], 4),
                    jnp.tile(gamma[0, BH:], 4)]).reshape(NC, 1, 128)

    chn_p = jnp.stack([chn_llr[:, :BH].reshape(N // 4, 128),
                       chn_llr[:, BH:].reshape(N // 4, 128)])
    zeros_m = jnp.zeros((M, BH), _f32)
    c2v_p = jnp.zeros((NC, _EROWS, 128), _f32)
    v2c_p = jnp.zeros((NC, _EROWS, 128), _f32)

    colg_p = _sccol_first(ecol2d, chn_p)

    outs = []
    for _t in range(T):
        v2c_p, sign_p, lth_p = _tcv(g2, v2c_p, c2v_p, colg_p)
        parg_p, ampg_p = _sch(sign_p, lth_p, erow2d, zeros_m)
        c2v_p = _tcb(g2, parg_p, ampg_p, sign_p, lth_p, c2v_p)
        if _t < T - 1:
            colg_p, out_p = _sccol(c2v_p, ecol2d, chn_p)
        else:
            out_p = _sccol_last(c2v_p, ecol2d, chn_p)
        outs.append(out_p)
    out = jnp.stack(outs)
    out = out.reshape(T, NC, N, BH)
    return jnp.concatenate([out[:, 0], out[:, 1]], axis=-1)
